# baseline (speedup 1.0000x reference)
import functools

import jax
import jax.numpy as jnp
from jax import lax
from jax.experimental import pallas as pl
from jax.experimental.pallas import tpu as pltpu
from jax.experimental.pallas import tpu_sc as plsc

L = 16
B = 512
C = 8
RCH = 8
NBUF = 2


def _sc_hist_kernel(nc, ns, w, h, n_img):
    mesh = plsc.VectorSubcoreMesh(core_axis_name="c", subcore_axis_name="s")
    slices = ns // (n_img // nc)
    srows = h // slices
    nch = srows // RCH
    cpx = RCH * w
    img_per_sc = n_img // nc

    @functools.partial(
        pl.kernel,
        out_type=jax.ShapeDtypeStruct((32, L), jnp.float32),
        mesh=mesh,
        compiler_params=pltpu.CompilerParams(needs_layout_passes=False),
        scratch_types=[
            *[pltpu.VMEM((C, RCH, w), jnp.float32)
              for _ in range(NBUF)],
            *[pltpu.VMEM((RCH, w), jnp.int32)
              for _ in range(NBUF)],
            pltpu.VMEM((C, 2 * B), jnp.float32),
            pltpu.VMEM((2 * B,), jnp.float32),
            pltpu.VMEM((2 * B,), jnp.float32),
            pltpu.VMEM((L,), jnp.float32),
            pltpu.VMEM_SHARED((ns, C, 2 * B), jnp.float32),
            *[pltpu.SemaphoreType.DMA for _ in range(2 * NBUF)],
        ],
    )
    def hist_kernel(p_hbm, t_hbm, out_hbm, *rest):
        pbufs = rest[0:NBUF]
        tbufs = rest[NBUF:2 * NBUF]
        hist = rest[2 * NBUF]
        lhist = rest[2 * NBUF + 1]
        tmp = rest[2 * NBUF + 2]
        res = rest[2 * NBUF + 3]
        acc = rest[2 * NBUF + 4]
        psems = rest[2 * NBUF + 5:2 * NBUF + 5 + NBUF]
        tsems = rest[2 * NBUF + 5 + NBUF:]

        cid = lax.axis_index("c")
        sid = lax.axis_index("s")
        wid = cid * ns + sid
        img_local = sid // C
        img = img_per_sc * cid + img_local
        sl = sid % slices

        zeros = jnp.zeros((L,), jnp.float32)
        ones = jnp.ones((L,), jnp.float32)
        fB = jnp.float32(B)

        trow0 = img * h + sl * srows

        def issue(ch, b):
            pltpu.async_copy(
                p_hbm.at[pl.ds(img * C, C),
                         pl.ds(sl * srows + ch * RCH, RCH), :],
                pbufs[b], psems[b])
            pltpu.async_copy(
                t_hbm.at[pl.ds(trow0 + ch * RCH, RCH), :], tbufs[b], tsems[b])

        def wait(b):
            pltpu.make_async_copy(
                p_hbm.at[pl.ds(0, C), pl.ds(0, RCH), :],
                pbufs[b], psems[b]).wait()
            pltpu.make_async_copy(
                t_hbm.at[pl.ds(0, RCH), :], tbufs[b], tsems[b]).wait()

        for b in range(NBUF):
            issue(b, b)

        for c in range(C):
            def zbody(j, _, _c=c):
                hist[_c, pl.ds(j * L, L)] = zeros
                return 0
            lax.fori_loop(0, (2 * B) // L, zbody, 0)

        wshift = w.bit_length() - 1
        tiny = jnp.float32(1e-4)
        cvecs = [jnp.full((L,), c, jnp.int32) for c in range(C)]

        def chunk_group(g, _):
            for b in range(NBUF):
                ch = g * NBUF + b
                wait(b)
                pbs = pbufs[b]
                tb = tbufs[b]

                @plsc.parallel_loop(0, cpx // L, step=1, unroll=4)
                def _3(k):
                    i = k * L
                    r = lax.shift_right_logical(i, wshift)
                    col = lax.bitwise_and(i, w - 1)
                    t = tb[r, pl.ds(col, L)]
                    for c in range(C):
                        p = jnp.maximum(pbs[c, r, pl.ds(col, L)], tiny)
                        bneg = (p * fB).astype(jnp.int32)
                        bpos = ((2.0 - p) * fB).astype(jnp.int32)
                        iy = jnp.where(t == c, bpos, bneg)
                        plsc.addupdate_scatter(hist, [cvecs[c], iy], ones)

                nxt = ch + NBUF

                @pl.when(nxt < nch)
                def _():
                    issue(nxt, b)
            return 0

        lax.fori_loop(0, nch // NBUF, chunk_group, 0)

        pltpu.sync_copy(hist, acc.at[sid])
        plsc.subcore_barrier()
        cown = sid % C
        base = img_local * slices
        pltpu.sync_copy(acc.at[base, cown], lhist)
        for s in range(1, slices):
            pltpu.sync_copy(acc.at[base + s, cown], tmp)

            def rbody(j, _, ):
                lhist[pl.ds(j * L, L)] = (
                    lhist[pl.ds(j * L, L)] + tmp[pl.ds(j * L, L)])
                return 0
            lax.fori_loop(0, (2 * B) // L, rbody, 0)

        nq = B // (4 * L)
        pq = 3 * B // (4 * L)

        def gbody(j, carry):
            gv, cv = carry
            hpc = lhist[pl.ds(B + j * L, L)]
            hnc = lhist[pl.ds(j * L, L)]
            mn = (j >= nq).astype(jnp.float32)
            mp = (j < pq).astype(jnp.float32)
            return (gv + hpc, cv + hnc * mn + hpc * mp)

        gv, cv = lax.fori_loop(0, B // L, gbody, (zeros, zeros))
        G = jnp.sum(gv)
        cnt25 = jnp.sum(cv)

        def sbody(j, carry):
            jacc, pc, kc = carry
            asc = B - L * (j + 1)
            hp = lax.rev(lhist[pl.ds(B + asc, L)], (0,))
            hn = lax.rev(lhist[pl.ds(asc, L)], (0,))
            n = hp + hn
            pv = pc + plsc.cumsum(hp)
            kv = kc + plsc.cumsum(n)
            jbin = jnp.where(
                kv > 0.0,
                1.0 - (G - pv) / jnp.maximum(G + kv - pv, 1.0),
                0.0,
            )
            return (jacc + jbin, pc + jnp.sum(hp), kc + jnp.sum(n))

        jacc, _, _ = lax.fori_loop(
            0, B // L, sbody, (zeros, jnp.float32(0.0), jnp.float32(0.0)))
        term = (jnp.sum(jacc) - 0.5) * jnp.float32(1.0 / B)

        lane = lax.iota(jnp.int32, L)
        res[...] = jnp.where(
            lane == 0, term,
            jnp.where(lane == 1, G, jnp.where(lane == 2, cnt25, 0.0)))
        pltpu.sync_copy(res, out_hbm.at[img * C + sid % C])

    return hist_kernel


def _finalize_body(res_ref, aux_ref, out_ref):
    res = res_ref[...]
    aux = aux_ref[...]
    loss_t = res[:, 0:1]
    g = res[:, 1:2]
    c25 = res[:, 2:3]
    active = (aux[:, 1:2] != 0.0) & ((g > 0.0) | (c25 > 0.0))
    flag = active.astype(jnp.float32)
    total = jnp.sum(loss_t * aux[:, 0:1] * flag)
    ne = jnp.sum(flag)
    out_ref[...] = (total / 4.0 / ne)[None, None]


def kernel(inputs, targets, classes_weights, tiles_weights, config):
    n, c_dim, h, w = inputs.shape

    info = plsc.get_sparse_core_info()

    p3 = inputs.reshape(n * c_dim, h, w)
    t2 = targets.reshape(n * h, w)

    res32 = _sc_hist_kernel(info.num_cores, info.num_subcores, w, h, n)(p3, t2)

    cw_full = jnp.tile(classes_weights, n)
    tw_full = jnp.repeat(tiles_weights, c_dim)
    aux = jnp.zeros((32, L), jnp.float32)
    aux = aux.at[:, 0].set(cw_full * tw_full)
    aux = aux.at[:, 1].set(cw_full)

    out = pl.pallas_call(
        _finalize_body,
        out_shape=jax.ShapeDtypeStruct((1, 1), jnp.float32),
    )(res32, aux)
    return out[0, 0]


# speedup vs baseline: 1.3122x; 1.3122x over previous
import functools

import jax
import jax.numpy as jnp
from jax import lax
from jax.experimental import pallas as pl
from jax.experimental.pallas import tpu as pltpu
from jax.experimental.pallas import tpu_sc as plsc

L = 16
B = 512
CS = 16384


def _sc_hist_kernel(nc, w, rows_per_term, nch):
    mesh = plsc.VectorSubcoreMesh(core_axis_name="c", subcore_axis_name="s")
    rows = CS // w

    @functools.partial(
        pl.kernel,
        out_type=jax.ShapeDtypeStruct((32, L), jnp.float32),
        mesh=mesh,
        compiler_params=pltpu.CompilerParams(needs_layout_passes=False),
        scratch_types=[
            pltpu.VMEM((rows, w), jnp.float32),
            pltpu.VMEM((rows, w), jnp.float32),
            pltpu.VMEM((rows, w), jnp.int32),
            pltpu.VMEM((rows, w), jnp.int32),
            pltpu.VMEM((2 * B,), jnp.float32),
            pltpu.VMEM((L,), jnp.float32),
            pltpu.SemaphoreType.DMA,
            pltpu.SemaphoreType.DMA,
            pltpu.SemaphoreType.DMA,
            pltpu.SemaphoreType.DMA,
        ],
    )
    def hist_kernel(p_hbm, t_hbm, out_hbm, p0, p1, t0, t1, hist, res,
                    ps0, ps1, ts0, ts1):
        cid = lax.axis_index("c")
        sid = lax.axis_index("s")
        wid = sid * nc + cid
        img = wid // 8
        cls = wid % 8

        pbufs = (p0, p1)
        tbufs = (t0, t1)
        psems = (ps0, ps1)
        tsems = (ts0, ts1)

        zeros = jnp.zeros((L,), jnp.float32)
        ones = jnp.ones((L,), jnp.float32)
        fB = jnp.float32(B)

        prow = wid * rows_per_term
        trow = img * rows_per_term

        def issue(ch, b):
            pltpu.async_copy(
                p_hbm.at[pl.ds(prow + ch * rows, rows), :], pbufs[b], psems[b])
            pltpu.async_copy(
                t_hbm.at[pl.ds(trow + ch * rows, rows), :], tbufs[b], tsems[b])

        def wait(b):
            pltpu.make_async_copy(
                p_hbm.at[pl.ds(0, rows), :], pbufs[b], psems[b]).wait()
            pltpu.make_async_copy(
                t_hbm.at[pl.ds(0, rows), :], tbufs[b], tsems[b]).wait()

        issue(0, 0)
        issue(1, 1)

        def zbody(j, _):
            hist[pl.ds(j * L, L)] = zeros
            return 0
        lax.fori_loop(0, (2 * B) // L, zbody, 0)

        def chunk_group(g, _):
            for b in range(2):
                ch = g * 2 + b
                wait(b)
                pb = pbufs[b]
                tb = tbufs[b]

                wshift = w.bit_length() - 1

                @plsc.parallel_loop(0, CS // L, step=1, unroll=16)
                def _3(k):
                    i = k * L
                    r = lax.shift_right_logical(i, wshift)
                    col = lax.bitwise_and(i, w - 1)
                    p = pb[r, pl.ds(col, L)]
                    t = tb[r, pl.ds(col, L)]
                    m = t == cls
                    v = jnp.where(m, 2.0 - p, p)
                    iy = (v * fB).astype(jnp.int32)
                    iy = jnp.minimum(iy, 2 * B - 1)
                    plsc.addupdate_scatter(hist, [iy], ones)

                nxt = ch + 2

                @pl.when(nxt < nch)
                def _():
                    issue(nxt, b)
            return 0

        lax.fori_loop(0, nch // 2, chunk_group, 0)

        nq = B // (4 * L)
        pq = 3 * B // (4 * L)

        def gbody(j, carry):
            g, c25 = carry
            s_hp = jnp.sum(hist[pl.ds(B + j * L, L)])
            s_hn = jnp.sum(hist[pl.ds(j * L, L)])
            c25 = c25 + jnp.where(j >= nq, s_hn, 0.0)
            c25 = c25 + jnp.where(j < pq, s_hp, 0.0)
            return (g + s_hp, c25)

        G, cnt25 = lax.fori_loop(
            0, B // L, gbody, (jnp.float32(0.0), jnp.float32(0.0)))

        def sbody(j, carry):
            jacc, pc, kc = carry
            asc = B - L * (j + 1)
            hp = lax.rev(hist[pl.ds(B + asc, L)], (0,))
            hn = lax.rev(hist[pl.ds(asc, L)], (0,))
            n = hp + hn
            pv = pc + plsc.cumsum(hp)
            kv = kc + plsc.cumsum(n)
            jbin = jnp.where(
                kv > 0.0,
                1.0 - (G - pv) / jnp.maximum(G + kv - pv, 1.0),
                0.0,
            )
            return (jacc + jbin, pc + jnp.sum(hp), kc + jnp.sum(n))

        jacc, _, _ = lax.fori_loop(
            0, B // L, sbody, (zeros, jnp.float32(0.0), jnp.float32(0.0)))
        term = (jnp.sum(jacc) - 0.5) * jnp.float32(1.0 / B)

        lane = lax.iota(jnp.int32, L)
        res[...] = jnp.where(
            lane == 0, term,
            jnp.where(lane == 1, G, jnp.where(lane == 2, cnt25, 0.0)))
        pltpu.sync_copy(res, out_hbm.at[wid])

    return hist_kernel


def _finalize_body(res_ref, aux_ref, out_ref):
    res = res_ref[...]
    aux = aux_ref[...]
    loss_t = res[:, 0:1]
    g = res[:, 1:2]
    c25 = res[:, 2:3]
    active = (aux[:, 1:2] != 0.0) & ((g > 0.0) | (c25 > 0.0))
    flag = active.astype(jnp.float32)
    total = jnp.sum(loss_t * aux[:, 0:1] * flag)
    ne = jnp.sum(flag)
    out_ref[...] = (total / 4.0 / ne)[None, None]


def kernel(inputs, targets, classes_weights, tiles_weights, config):
    n, c_dim, h, w = inputs.shape
    hw = h * w
    nch = hw // CS

    info = plsc.get_sparse_core_info()
    nc = info.num_cores

    p2 = inputs.reshape(n * c_dim * h, w)
    t2 = targets.reshape(n * h, w)

    res32 = _sc_hist_kernel(nc, w, h, nch)(p2, t2)

    cw_full = jnp.tile(classes_weights, n)
    tw_full = jnp.repeat(tiles_weights, c_dim)
    aux = jnp.zeros((32, L), jnp.float32)
    aux = aux.at[:, 0].set(cw_full * tw_full)
    aux = aux.at[:, 1].set(cw_full)

    out = pl.pallas_call(
        _finalize_body,
        out_shape=jax.ShapeDtypeStruct((1, 1), jnp.float32),
    )(res32, aux)
    return out[0, 0]


# speedup vs baseline: 1.3154x; 1.0024x over previous
import functools

import jax
import jax.numpy as jnp
from jax import lax
from jax.experimental import pallas as pl
from jax.experimental.pallas import tpu as pltpu
from jax.experimental.pallas import tpu_sc as plsc

L = 16
B = 512
CS = 16384


def _sc_hist_kernel(nc, w, rows_per_term, nch):
    mesh = plsc.VectorSubcoreMesh(core_axis_name="c", subcore_axis_name="s")
    rows = CS // w

    @functools.partial(
        pl.kernel,
        out_type=jax.ShapeDtypeStruct((32, L), jnp.float32),
        mesh=mesh,
        compiler_params=pltpu.CompilerParams(needs_layout_passes=False),
        scratch_types=[
            pltpu.VMEM((rows, w), jnp.float32),
            pltpu.VMEM((rows, w), jnp.float32),
            pltpu.VMEM((rows, w), jnp.int32),
            pltpu.VMEM((rows, w), jnp.int32),
            pltpu.VMEM((2 * B,), jnp.float32),
            pltpu.VMEM((L,), jnp.float32),
            pltpu.SemaphoreType.DMA,
            pltpu.SemaphoreType.DMA,
            pltpu.SemaphoreType.DMA,
            pltpu.SemaphoreType.DMA,
        ],
    )
    def hist_kernel(p_hbm, t_hbm, out_hbm, p0, p1, t0, t1, hist, res,
                    ps0, ps1, ts0, ts1):
        cid = lax.axis_index("c")
        sid = lax.axis_index("s")
        wid = sid * nc + cid
        img = wid // 8
        cls = wid % 8

        pbufs = (p0, p1)
        tbufs = (t0, t1)
        psems = (ps0, ps1)
        tsems = (ts0, ts1)

        zeros = jnp.zeros((L,), jnp.float32)
        ones = jnp.ones((L,), jnp.float32)
        fB = jnp.float32(B)

        prow = wid * rows_per_term
        trow = img * rows_per_term

        def issue(ch, b):
            pltpu.async_copy(
                p_hbm.at[pl.ds(prow + ch * rows, rows), :], pbufs[b], psems[b])
            pltpu.async_copy(
                t_hbm.at[pl.ds(trow + ch * rows, rows), :], tbufs[b], tsems[b])

        def wait(b):
            pltpu.make_async_copy(
                p_hbm.at[pl.ds(0, rows), :], pbufs[b], psems[b]).wait()
            pltpu.make_async_copy(
                t_hbm.at[pl.ds(0, rows), :], tbufs[b], tsems[b]).wait()

        issue(0, 0)
        issue(1, 1)

        def zbody(j, _):
            hist[pl.ds(j * L, L)] = zeros
            return 0
        lax.fori_loop(0, (2 * B) // L, zbody, 0)

        def chunk_group(g, _):
            for b in range(2):
                ch = g * 2 + b
                wait(b)
                pb = pbufs[b]
                tb = tbufs[b]

                wshift = w.bit_length() - 1

                @plsc.parallel_loop(0, CS // L, step=1, unroll=16)
                def _3(k):
                    i = k * L
                    r = lax.shift_right_logical(i, wshift)
                    col = lax.bitwise_and(i, w - 1)
                    p = pb[r, pl.ds(col, L)]
                    t = tb[r, pl.ds(col, L)]
                    m = t == cls
                    v = jnp.where(m, 2.0 - p, p)
                    iy = (v * fB).astype(jnp.int32)
                    iy = jnp.minimum(iy, 2 * B - 1)
                    plsc.addupdate_scatter(hist, [iy], ones)

                nxt = ch + 2

                @pl.when(nxt < nch)
                def _():
                    issue(nxt, b)
            return 0

        lax.fori_loop(0, nch // 2, chunk_group, 0)

        nq = B // (4 * L)
        pq = 3 * B // (4 * L)

        def gbody(j, carry):
            gv, cv = carry
            hpc = hist[pl.ds(B + j * L, L)]
            hnc = hist[pl.ds(j * L, L)]
            mn = (j >= nq).astype(jnp.float32)
            mp = (j < pq).astype(jnp.float32)
            return (gv + hpc, cv + hnc * mn + hpc * mp)

        gv, cv = lax.fori_loop(0, B // L, gbody, (zeros, zeros))
        G = jnp.sum(gv)
        cnt25 = jnp.sum(cv)

        def sbody(j, carry):
            jacc, pc, kc = carry
            asc = B - L * (j + 1)
            hp = lax.rev(hist[pl.ds(B + asc, L)], (0,))
            hn = lax.rev(hist[pl.ds(asc, L)], (0,))
            n = hp + hn
            pv = pc + plsc.cumsum(hp)
            kv = kc + plsc.cumsum(n)
            jbin = jnp.where(
                kv > 0.0,
                1.0 - (G - pv) / jnp.maximum(G + kv - pv, 1.0),
                0.0,
            )
            return (jacc + jbin, pc + jnp.sum(hp), kc + jnp.sum(n))

        jacc, _, _ = lax.fori_loop(
            0, B // L, sbody, (zeros, jnp.float32(0.0), jnp.float32(0.0)))
        term = (jnp.sum(jacc) - 0.5) * jnp.float32(1.0 / B)

        lane = lax.iota(jnp.int32, L)
        res[...] = jnp.where(
            lane == 0, term,
            jnp.where(lane == 1, G, jnp.where(lane == 2, cnt25, 0.0)))
        pltpu.sync_copy(res, out_hbm.at[wid])

    return hist_kernel


def _finalize_body(res_ref, aux_ref, out_ref):
    res = res_ref[...]
    aux = aux_ref[...]
    loss_t = res[:, 0:1]
    g = res[:, 1:2]
    c25 = res[:, 2:3]
    active = (aux[:, 1:2] != 0.0) & ((g > 0.0) | (c25 > 0.0))
    flag = active.astype(jnp.float32)
    total = jnp.sum(loss_t * aux[:, 0:1] * flag)
    ne = jnp.sum(flag)
    out_ref[...] = (total / 4.0 / ne)[None, None]


def kernel(inputs, targets, classes_weights, tiles_weights, config):
    n, c_dim, h, w = inputs.shape
    hw = h * w
    nch = hw // CS

    info = plsc.get_sparse_core_info()
    nc = info.num_cores

    p2 = inputs.reshape(n * c_dim * h, w)
    t2 = targets.reshape(n * h, w)

    res32 = _sc_hist_kernel(nc, w, h, nch)(p2, t2)

    cw_full = jnp.tile(classes_weights, n)
    tw_full = jnp.repeat(tiles_weights, c_dim)
    aux = jnp.zeros((32, L), jnp.float32)
    aux = aux.at[:, 0].set(cw_full * tw_full)
    aux = aux.at[:, 1].set(cw_full)

    out = pl.pallas_call(
        _finalize_body,
        out_shape=jax.ShapeDtypeStruct((1, 1), jnp.float32),
    )(res32, aux)
    return out[0, 0]
